# baseline (speedup 1.0000x reference)
import jax
import jax.numpy as jnp
from jax import lax
from jax.experimental import pallas as pl
from jax.experimental.pallas import tpu as pltpu
from jax.experimental.pallas import tpu_sc as plsc

N_NODES = 10000
N_EDGES = 320000
EPS = 1e-5

NC = 2
NS = 16
CHUNK = 128
EDGES_PAD = 327680
K_TOTAL = EDGES_PAD // (NS * CHUNK)
K0 = 160
K1 = K_TOTAL - K0
NPAD = 10240
ROWS_PER_TILE = NPAD // NS
DUMMY = N_NODES
IDX_BLK = 16
K_CNT = K_TOTAL // NC
ZERO_STEPS = ROWS_PER_TILE // CHUNK

_MESH = plsc.VectorSubcoreMesh(core_axis_name="c", subcore_axis_name="s")


def _zero_acc_slice(zrow, stage, acc, row0):
  pltpu.sync_copy(zrow, stage)
  for r in range(ZERO_STEPS):
    pltpu.sync_copy(stage, acc.at[pl.ds(row0 + r * CHUNK, CHUNK)])


def _copy_out_slice(acc, stage, out, row0, out0):
  for r in range(ZERO_STEPS):
    pltpu.sync_copy(acc.at[pl.ds(row0 + r * CHUNK, CHUNK)], stage)
    pltpu.sync_copy(stage, out.at[pl.ds(out0 + r * CHUNK, CHUNK)])


def _sc_agg_body(table, src_r, dst_r, zrow, sums_out,
                 src_v, dst_v, rows_a, rows_b, acc, sem_a, sem_b):
  c = lax.axis_index("c")
  s = lax.axis_index("s")
  row0 = s * ROWS_PER_TILE
  _zero_acc_slice(zrow, rows_a, acc, row0)
  plsc.subcore_barrier()

  bufs = (rows_a, rows_b)
  sems = (sem_a, sem_b)
  tile_base = jnp.where(c == 0, s * K0, NS * K0 + s * K1)
  n_blks = jnp.where(c == 0, K0 // IDX_BLK, K1 // IDX_BLK)

  def blk(b, carry):
    base = tile_base + b * IDX_BLK
    pltpu.sync_copy(src_r.at[pl.ds(base, IDX_BLK)], src_v)
    pltpu.sync_copy(dst_r.at[pl.ds(base, IDX_BLK)], dst_v)
    cps = [None] * IDX_BLK
    cps[0] = pltpu.async_copy(table.at[src_v.at[0]], bufs[0], sems[0])
    for j in range(IDX_BLK):
      if j + 1 < IDX_BLK:
        p = (j + 1) % 2
        cps[j + 1] = pltpu.async_copy(table.at[src_v.at[j + 1]], bufs[p], sems[p])
      cps[j].wait()
      pltpu.sync_copy(bufs[j % 2], acc.at[dst_v.at[j]], add=True)
    return carry

  lax.fori_loop(0, n_blks, blk, 0)
  plsc.subcore_barrier()
  _copy_out_slice(acc, rows_a, sums_out, row0, c * NPAD + row0)


_sc_agg = pl.kernel(
    _sc_agg_body,
    out_type=jax.ShapeDtypeStruct((NC * NPAD, 128), jnp.float32),
    mesh=_MESH,
    scratch_types=[
        pltpu.VMEM((IDX_BLK, CHUNK), jnp.int32),
        pltpu.VMEM((IDX_BLK, CHUNK), jnp.int32),
        pltpu.VMEM((CHUNK, 128), jnp.float32),
        pltpu.VMEM((CHUNK, 128), jnp.float32),
        pltpu.VMEM_SHARED((NPAD, 128), jnp.float32),
        pltpu.SemaphoreType.DMA,
        pltpu.SemaphoreType.DMA,
    ],
    name="sc_segsum")


def _sc_cnt_body(dst_r, zrow, ones_hbm, cnts_out, dst_v, stage_v, ones_v, acc):
  c = lax.axis_index("c")
  s = lax.axis_index("s")
  wid = c * NS + s
  row0 = s * ROWS_PER_TILE
  _zero_acc_slice(zrow, stage_v, acc, row0)
  pltpu.sync_copy(ones_hbm, ones_v)
  plsc.subcore_barrier()

  def blk(b, carry):
    base = wid * K_CNT + b * IDX_BLK
    pltpu.sync_copy(dst_r.at[pl.ds(base, IDX_BLK)], dst_v)
    for j in range(IDX_BLK):
      pltpu.sync_copy(ones_v, acc.at[dst_v.at[j]], add=True)
    return carry

  lax.fori_loop(0, K_CNT // IDX_BLK, blk, 0)
  plsc.subcore_barrier()
  _copy_out_slice(acc, stage_v, cnts_out, row0, c * NPAD + row0)


_sc_cnt = pl.kernel(
    _sc_cnt_body,
    out_type=jax.ShapeDtypeStruct((NC * NPAD, 128), jnp.float32),
    mesh=_MESH,
    scratch_types=[
        pltpu.VMEM((IDX_BLK, CHUNK), jnp.int32),
        pltpu.VMEM((CHUNK, 128), jnp.float32),
        pltpu.VMEM((CHUNK, 128), jnp.float32),
        pltpu.VMEM_SHARED((NPAD, 128), jnp.float32),
    ],
    name="sc_counts")


ROW_BLK = 640
GRID = NPAD // ROW_BLK


def _ln_relu(pre, g, b):
  mu = jnp.mean(pre, axis=-1, keepdims=True)
  d = pre - mu
  var = jnp.mean(d * d, axis=-1, keepdims=True)
  return jax.nn.relu(d * lax.rsqrt(var + EPS) * g + b)


def _tc1_body(x_ref, sums_ref, cnts_ref, wl1, bl1, wr1, g1, beta1, w1, bw1,
              w2, bw2, wl2, wr2, bl2, y2_ref, z2_ref):
  sum1 = sums_ref[0] + sums_ref[1]
  cnt = cnts_ref[0, :, 0:1] + cnts_ref[1, :, 0:1]
  mean1 = sum1 * (1.0 / jnp.maximum(cnt, 1.0))
  pre = (jnp.dot(mean1, wl1[...], preferred_element_type=jnp.float32)
         + jnp.dot(x_ref[...], wr1[...], preferred_element_type=jnp.float32)
         + bl1[...])
  h = _ln_relu(pre, g1[...], beta1[...])
  h = jax.nn.relu(jnp.dot(h, w1[...], preferred_element_type=jnp.float32) + bw1[...])
  h = jax.nn.relu(jnp.dot(h, w2[...], preferred_element_type=jnp.float32) + bw2[...])
  y2_ref[...] = jnp.dot(h, wl2[...], preferred_element_type=jnp.float32)
  z2_ref[...] = jnp.dot(h, wr2[...], preferred_element_type=jnp.float32) + bl2[...]


def _tc2_body(sums_ref, cnts_ref, z2_ref, g2, beta2, w3, bw3, w4, bw4, out_ref):
  sum2 = sums_ref[0] + sums_ref[1]
  cnt = cnts_ref[0, :, 0:1] + cnts_ref[1, :, 0:1]
  mean2 = sum2 * (1.0 / jnp.maximum(cnt, 1.0))
  h = _ln_relu(mean2 + z2_ref[...], g2[...], beta2[...])
  h = jax.nn.relu(jnp.dot(h, w3[...], preferred_element_type=jnp.float32) + bw3[...])
  out_ref[...] = jnp.dot(h, w4[...], preferred_element_type=jnp.float32) + bw4[...]


def _row_spec(width):
  return pl.BlockSpec((ROW_BLK, width), lambda i: (i, 0))


def _part_spec(width):
  return pl.BlockSpec((NC, ROW_BLK, width), lambda i: (0, i, 0))


def _full_spec(shape):
  return pl.BlockSpec(shape, lambda i: tuple(0 for _ in shape))


def kernel(x, edge_index, Wl1, bl1, Wr1, g1, beta1, W1, bW1, W2, bW2,
           Wl2, bl2, Wr2, g2, beta2, W3, bW3, W4, bW4):
  f32 = jnp.float32
  x = x.astype(f32)
  src = edge_index[0].astype(jnp.int32)
  dst = edge_index[1].astype(jnp.int32)
  pad = EDGES_PAD - N_EDGES
  src_r = jnp.concatenate([src, jnp.zeros((pad,), jnp.int32)]).reshape(NS * K_TOTAL, CHUNK)
  dst_r = jnp.concatenate([dst, jnp.full((pad,), DUMMY, jnp.int32)]).reshape(NS * K_TOTAL, CHUNK)
  x_pad = jnp.zeros((NPAD, 128), f32).at[:N_NODES].set(x)
  zrow = jnp.zeros((CHUNK, 128), f32)
  ones = jnp.ones((CHUNK, 128), f32)

  cnts = _sc_cnt(dst_r, zrow, ones).reshape(NC, NPAD, 128)
  sums1 = _sc_agg(x_pad, src_r, dst_r, zrow).reshape(NC, NPAD, 128)

  grid = (GRID,)
  y2, z2 = pl.pallas_call(
      _tc1_body,
      grid=grid,
      in_specs=[
          _row_spec(128),
          _part_spec(128),
          _part_spec(128),
          _full_spec((128, 256)),
          _full_spec((1, 256)),
          _full_spec((128, 256)),
          _full_spec((1, 256)),
          _full_spec((1, 256)),
          _full_spec((256, 512)),
          _full_spec((1, 512)),
          _full_spec((512, 256)),
          _full_spec((1, 256)),
          _full_spec((256, 128)),
          _full_spec((256, 128)),
          _full_spec((1, 128)),
      ],
      out_specs=[_row_spec(128), _row_spec(128)],
      out_shape=[jax.ShapeDtypeStruct((NPAD, 128), f32),
                 jax.ShapeDtypeStruct((NPAD, 128), f32)],
  )(x_pad, sums1, cnts, Wl1.T, bl1[None, :], Wr1.T, g1[None, :],
    beta1[None, :], W1.T, bW1[None, :], W2.T, bW2[None, :], Wl2.T, Wr2.T,
    bl2[None, :])

  sums2 = _sc_agg(y2, src_r, dst_r, zrow).reshape(NC, NPAD, 128)

  (out,) = pl.pallas_call(
      _tc2_body,
      grid=grid,
      in_specs=[
          _part_spec(128),
          _part_spec(128),
          _row_spec(128),
          _full_spec((1, 128)),
          _full_spec((1, 128)),
          _full_spec((128, 256)),
          _full_spec((1, 256)),
          _full_spec((256, 128)),
          _full_spec((1, 128)),
      ],
      out_specs=[_row_spec(128)],
      out_shape=[jax.ShapeDtypeStruct((NPAD, 128), f32)],
  )(sums2, cnts, z2, g2[None, :], beta2[None, :], W3.T, bW3[None, :],
    W4.T, bW4[None, :])

  return out[:N_NODES]


# speedup vs baseline: 3.3948x; 3.3948x over previous
import jax
import jax.numpy as jnp
from jax import lax
from jax.experimental import pallas as pl
from jax.experimental.pallas import tpu as pltpu
from jax.experimental.pallas import tpu_sc as plsc

N_NODES = 10000
N_EDGES = 320000
EPS = 1e-5

NC = 2
NS = 16
CHUNK = 128
EDGES_PAD = 327680
K_TOTAL = EDGES_PAD // (NS * CHUNK)
K0 = 80
K1 = K_TOTAL - K0
NPAD = 10240
ROWS_PER_TILE = NPAD // NS
DUMMY = N_NODES
IDX_BLK = 16
K_CNT = K_TOTAL // NC
ZERO_STEPS = ROWS_PER_TILE // CHUNK

_MESH = plsc.VectorSubcoreMesh(core_axis_name="c", subcore_axis_name="s")


def _zero_acc_slice(zrow, stage, acc, row0):
  pltpu.sync_copy(zrow, stage)
  for r in range(ZERO_STEPS):
    pltpu.sync_copy(stage, acc.at[pl.ds(row0 + r * CHUNK, CHUNK)])


def _copy_out_slice(acc, stage, out, row0, out0):
  for r in range(ZERO_STEPS):
    pltpu.sync_copy(acc.at[pl.ds(row0 + r * CHUNK, CHUNK)], stage)
    pltpu.sync_copy(stage, out.at[pl.ds(out0 + r * CHUNK, CHUNK)])


def _sc_agg_body(table, src_r, dst_r, zrow, sums_out,
                 src_v, dst_v, rows_a, rows_b, acc, sem_a, sem_b):
  c = lax.axis_index("c")
  s = lax.axis_index("s")
  row0 = s * ROWS_PER_TILE
  _zero_acc_slice(zrow, rows_a, acc, row0)
  plsc.subcore_barrier()

  bufs = (rows_a, rows_b)
  sems = (sem_a, sem_b)
  tile_base = jnp.where(c == 0, s * K0, NS * K0 + s * K1)
  n_blks = jnp.where(c == 0, K0 // IDX_BLK, K1 // IDX_BLK)

  def blk(b, carry):
    base = tile_base + b * IDX_BLK
    pltpu.sync_copy(src_r.at[pl.ds(base, IDX_BLK)], src_v)
    pltpu.sync_copy(dst_r.at[pl.ds(base, IDX_BLK)], dst_v)
    cps = [None] * IDX_BLK
    cps[0] = pltpu.async_copy(table.at[src_v.at[0]], bufs[0], sems[0])
    for j in range(IDX_BLK):
      if j + 1 < IDX_BLK:
        p = (j + 1) % 2
        cps[j + 1] = pltpu.async_copy(table.at[src_v.at[j + 1]], bufs[p], sems[p])
      cps[j].wait()
      pltpu.sync_copy(bufs[j % 2], acc.at[dst_v.at[j]], add=True)
    return carry

  lax.fori_loop(0, n_blks, blk, 0)
  plsc.subcore_barrier()
  _copy_out_slice(acc, rows_a, sums_out, row0, c * NPAD + row0)


_sc_agg = pl.kernel(
    _sc_agg_body,
    out_type=jax.ShapeDtypeStruct((NC * NPAD, 128), jnp.float32),
    mesh=_MESH,
    scratch_types=[
        pltpu.VMEM((IDX_BLK, CHUNK), jnp.int32),
        pltpu.VMEM((IDX_BLK, CHUNK), jnp.int32),
        pltpu.VMEM((CHUNK, 128), jnp.float32),
        pltpu.VMEM((CHUNK, 128), jnp.float32),
        pltpu.VMEM_SHARED((NPAD, 128), jnp.float32),
        pltpu.SemaphoreType.DMA,
        pltpu.SemaphoreType.DMA,
    ],
    name="sc_segsum")


def _sc_cnt_body(dst_r, zrow, ones_hbm, cnts_out, dst_v, stage_v, ones_v, acc):
  c = lax.axis_index("c")
  s = lax.axis_index("s")
  wid = c * NS + s
  row0 = s * ROWS_PER_TILE
  _zero_acc_slice(zrow, stage_v, acc, row0)
  pltpu.sync_copy(ones_hbm, ones_v)
  plsc.subcore_barrier()

  def blk(b, carry):
    base = wid * K_CNT + b * IDX_BLK
    pltpu.sync_copy(dst_r.at[pl.ds(base, IDX_BLK)], dst_v)
    for j in range(IDX_BLK):
      pltpu.sync_copy(ones_v, acc.at[dst_v.at[j]], add=True)
    return carry

  lax.fori_loop(0, K_CNT // IDX_BLK, blk, 0)
  plsc.subcore_barrier()
  _copy_out_slice(acc, stage_v, cnts_out, row0, c * NPAD + row0)


_sc_cnt = pl.kernel(
    _sc_cnt_body,
    out_type=jax.ShapeDtypeStruct((NC * NPAD, 128), jnp.float32),
    mesh=_MESH,
    scratch_types=[
        pltpu.VMEM((IDX_BLK, CHUNK), jnp.int32),
        pltpu.VMEM((CHUNK, 128), jnp.float32),
        pltpu.VMEM((CHUNK, 128), jnp.float32),
        pltpu.VMEM_SHARED((NPAD, 128), jnp.float32),
    ],
    name="sc_counts")


ROW_BLK = 640
GRID = NPAD // ROW_BLK


def _ln_relu(pre, g, b):
  mu = jnp.mean(pre, axis=-1, keepdims=True)
  d = pre - mu
  var = jnp.mean(d * d, axis=-1, keepdims=True)
  return jax.nn.relu(d * lax.rsqrt(var + EPS) * g + b)


def _tc1_body(x_ref, sums_ref, cnts_ref, wl1, bl1, wr1, g1, beta1, w1, bw1,
              w2, bw2, wl2, wr2, bl2, y2_ref, z2_ref):
  sum1 = sums_ref[0] + sums_ref[1]
  cnt = cnts_ref[0, :, 0:1] + cnts_ref[1, :, 0:1]
  mean1 = sum1 * (1.0 / jnp.maximum(cnt, 1.0))
  pre = (jnp.dot(mean1, wl1[...], preferred_element_type=jnp.float32)
         + jnp.dot(x_ref[...], wr1[...], preferred_element_type=jnp.float32)
         + bl1[...])
  h = _ln_relu(pre, g1[...], beta1[...])
  h = jax.nn.relu(jnp.dot(h, w1[...], preferred_element_type=jnp.float32) + bw1[...])
  h = jax.nn.relu(jnp.dot(h, w2[...], preferred_element_type=jnp.float32) + bw2[...])
  y2_ref[...] = jnp.dot(h, wl2[...], preferred_element_type=jnp.float32)
  z2_ref[...] = jnp.dot(h, wr2[...], preferred_element_type=jnp.float32) + bl2[...]


def _tc2_body(sums_ref, cnts_ref, z2_ref, g2, beta2, w3, bw3, w4, bw4, out_ref):
  sum2 = sums_ref[0] + sums_ref[1]
  cnt = cnts_ref[0, :, 0:1] + cnts_ref[1, :, 0:1]
  mean2 = sum2 * (1.0 / jnp.maximum(cnt, 1.0))
  h = _ln_relu(mean2 + z2_ref[...], g2[...], beta2[...])
  h = jax.nn.relu(jnp.dot(h, w3[...], preferred_element_type=jnp.float32) + bw3[...])
  out_ref[...] = jnp.dot(h, w4[...], preferred_element_type=jnp.float32) + bw4[...]


def _row_spec(width):
  return pl.BlockSpec((ROW_BLK, width), lambda i: (i, 0))


def _part_spec(width):
  return pl.BlockSpec((NC, ROW_BLK, width), lambda i: (0, i, 0))


def _full_spec(shape):
  return pl.BlockSpec(shape, lambda i: tuple(0 for _ in shape))


def kernel(x, edge_index, Wl1, bl1, Wr1, g1, beta1, W1, bW1, W2, bW2,
           Wl2, bl2, Wr2, g2, beta2, W3, bW3, W4, bW4):
  f32 = jnp.float32
  x = x.astype(f32)
  src = edge_index[0].astype(jnp.int32)
  dst = edge_index[1].astype(jnp.int32)
  pad = EDGES_PAD - N_EDGES
  pad_idx = jnp.arange(pad, dtype=jnp.int32)
  src_pad = (pad_idx * 997) % N_NODES
  dst_pad = DUMMY + (pad_idx % (NPAD - N_NODES))
  src_r = jnp.concatenate([src, src_pad]).reshape(NS * K_TOTAL, CHUNK)
  dst_r = jnp.concatenate([dst, dst_pad]).reshape(NS * K_TOTAL, CHUNK)
  x_pad = jnp.zeros((NPAD, 128), f32).at[:N_NODES].set(x)
  zrow = jnp.zeros((CHUNK, 128), f32)
  ones = jnp.ones((CHUNK, 128), f32)

  cnts = _sc_cnt(dst_r, zrow, ones).reshape(NC, NPAD, 128)
  sums1 = _sc_agg(x_pad, src_r, dst_r, zrow).reshape(NC, NPAD, 128)

  grid = (GRID,)
  y2, z2 = pl.pallas_call(
      _tc1_body,
      grid=grid,
      in_specs=[
          _row_spec(128),
          _part_spec(128),
          _part_spec(128),
          _full_spec((128, 256)),
          _full_spec((1, 256)),
          _full_spec((128, 256)),
          _full_spec((1, 256)),
          _full_spec((1, 256)),
          _full_spec((256, 512)),
          _full_spec((1, 512)),
          _full_spec((512, 256)),
          _full_spec((1, 256)),
          _full_spec((256, 128)),
          _full_spec((256, 128)),
          _full_spec((1, 128)),
      ],
      out_specs=[_row_spec(128), _row_spec(128)],
      out_shape=[jax.ShapeDtypeStruct((NPAD, 128), f32),
                 jax.ShapeDtypeStruct((NPAD, 128), f32)],
  )(x_pad, sums1, cnts, Wl1.T, bl1[None, :], Wr1.T, g1[None, :],
    beta1[None, :], W1.T, bW1[None, :], W2.T, bW2[None, :], Wl2.T, Wr2.T,
    bl2[None, :])

  sums2 = _sc_agg(y2, src_r, dst_r, zrow).reshape(NC, NPAD, 128)

  (out,) = pl.pallas_call(
      _tc2_body,
      grid=grid,
      in_specs=[
          _part_spec(128),
          _part_spec(128),
          _row_spec(128),
          _full_spec((1, 128)),
          _full_spec((1, 128)),
          _full_spec((128, 256)),
          _full_spec((1, 256)),
          _full_spec((256, 128)),
          _full_spec((1, 128)),
      ],
      out_specs=[_row_spec(128)],
      out_shape=[jax.ShapeDtypeStruct((NPAD, 128), f32)],
  )(sums2, cnts, z2, g2[None, :], beta2[None, :], W3.T, bW3[None, :],
    W4.T, bW4[None, :])

  return out[:N_NODES]


# speedup vs baseline: 3.4807x; 1.0253x over previous
import jax
import jax.numpy as jnp
from jax import lax
from jax.experimental import pallas as pl
from jax.experimental.pallas import tpu as pltpu
from jax.experimental.pallas import tpu_sc as plsc

N_NODES = 10000
N_EDGES = 320000
EPS = 1e-5

NC = 2
NS = 16
CHUNK = 128
EDGES_PAD = 327680
K_TOTAL = EDGES_PAD // (NS * CHUNK)
K0 = 80
K1 = K_TOTAL - K0
NPAD = 10240
ROWS_PER_TILE = NPAD // NS
DUMMY = N_NODES
IDX_BLK = 16
K_CNT = K_TOTAL // NC
ZERO_STEPS = ROWS_PER_TILE // CHUNK

_MESH = plsc.VectorSubcoreMesh(core_axis_name="c", subcore_axis_name="s")


def _zero_acc_slice(zrow, stage, acc, row0):
  pltpu.sync_copy(zrow, stage)
  for r in range(ZERO_STEPS):
    pltpu.sync_copy(stage, acc.at[pl.ds(row0 + r * CHUNK, CHUNK)])


def _copy_out_slice(acc, stage, out, row0, out0):
  for r in range(ZERO_STEPS):
    pltpu.sync_copy(acc.at[pl.ds(row0 + r * CHUNK, CHUNK)], stage)
    pltpu.sync_copy(stage, out.at[pl.ds(out0 + r * CHUNK, CHUNK)])


def _sc_agg_body(table, src_r, dst_r, zrow, sums_out,
                 src_v, dst_v, rows_a, rows_b, acc, sem_a, sem_b):
  c = lax.axis_index("c")
  s = lax.axis_index("s")
  row0 = s * ROWS_PER_TILE
  _zero_acc_slice(zrow, rows_a, acc, row0)
  plsc.subcore_barrier()

  bufs = (rows_a, rows_b)
  sems = (sem_a, sem_b)
  tile_base = jnp.where(c == 0, s * K0, NS * K0 + s * K1)
  n_blks = jnp.where(c == 0, K0 // IDX_BLK, K1 // IDX_BLK)

  def blk(b, carry):
    base = tile_base + b * IDX_BLK
    pltpu.sync_copy(src_r.at[pl.ds(base, IDX_BLK)], src_v)
    pltpu.sync_copy(dst_r.at[pl.ds(base, IDX_BLK)], dst_v)
    cps = [None] * IDX_BLK
    cps[0] = pltpu.async_copy(table.at[src_v.at[0]], bufs[0], sems[0])
    for j in range(IDX_BLK):
      if j + 1 < IDX_BLK:
        p = (j + 1) % 2
        cps[j + 1] = pltpu.async_copy(table.at[src_v.at[j + 1]], bufs[p], sems[p])
      cps[j].wait()
      pltpu.sync_copy(bufs[j % 2], acc.at[dst_v.at[j]], add=True)
    return carry

  lax.fori_loop(0, n_blks, blk, 0)
  plsc.subcore_barrier()
  _copy_out_slice(acc, rows_a, sums_out, row0, c * NPAD + row0)


_sc_agg = pl.kernel(
    _sc_agg_body,
    out_type=jax.ShapeDtypeStruct((NC * NPAD, 128), jnp.float32),
    mesh=_MESH,
    scratch_types=[
        pltpu.VMEM((IDX_BLK, CHUNK), jnp.int32),
        pltpu.VMEM((IDX_BLK, CHUNK), jnp.int32),
        pltpu.VMEM((CHUNK, 128), jnp.float32),
        pltpu.VMEM((CHUNK, 128), jnp.float32),
        pltpu.VMEM_SHARED((NPAD, 128), jnp.float32),
        pltpu.SemaphoreType.DMA,
        pltpu.SemaphoreType.DMA,
    ],
    name="sc_segsum")


def _sc_cnt_body(dst_r, zrow, ones_hbm, cnts_out, dst_v, stage_v, ones_v, acc):
  c = lax.axis_index("c")
  s = lax.axis_index("s")
  wid = c * NS + s
  row0 = s * ROWS_PER_TILE
  _zero_acc_slice(zrow, stage_v, acc, row0)
  pltpu.sync_copy(ones_hbm, ones_v)
  plsc.subcore_barrier()

  def blk(b, carry):
    base = wid * K_CNT + b * IDX_BLK
    pltpu.sync_copy(dst_r.at[pl.ds(base, IDX_BLK)], dst_v)
    for j in range(IDX_BLK):
      pltpu.sync_copy(ones_v, acc.at[dst_v.at[j]], add=True)
    return carry

  lax.fori_loop(0, K_CNT // IDX_BLK, blk, 0)
  plsc.subcore_barrier()
  _copy_out_slice(acc, stage_v, cnts_out, row0, c * NPAD + row0)


_sc_cnt = pl.kernel(
    _sc_cnt_body,
    out_type=jax.ShapeDtypeStruct((NC * NPAD, 128), jnp.float32),
    mesh=_MESH,
    scratch_types=[
        pltpu.VMEM((IDX_BLK, CHUNK), jnp.int32),
        pltpu.VMEM((CHUNK, 128), jnp.float32),
        pltpu.VMEM((CHUNK, 128), jnp.float32),
        pltpu.VMEM_SHARED((NPAD, 128), jnp.float32),
    ],
    name="sc_counts")


ROW_BLK = 1280
GRID = NPAD // ROW_BLK


def _ln_relu(pre, g, b):
  mu = jnp.mean(pre, axis=-1, keepdims=True)
  d = pre - mu
  var = jnp.mean(d * d, axis=-1, keepdims=True)
  return jax.nn.relu(d * lax.rsqrt(var + EPS) * g + b)


def _tc1_body(x_ref, sums_ref, cnts_ref, wl1, bl1, wr1, g1, beta1, w1, bw1,
              w2, bw2, wl2, wr2, bl2, y2_ref, z2_ref):
  sum1 = sums_ref[0] + sums_ref[1]
  cnt = cnts_ref[0, :, 0:1] + cnts_ref[1, :, 0:1]
  mean1 = sum1 * (1.0 / jnp.maximum(cnt, 1.0))
  pre = (jnp.dot(mean1, wl1[...], preferred_element_type=jnp.float32)
         + jnp.dot(x_ref[...], wr1[...], preferred_element_type=jnp.float32)
         + bl1[...])
  h = _ln_relu(pre, g1[...], beta1[...])
  h = jax.nn.relu(jnp.dot(h, w1[...], preferred_element_type=jnp.float32) + bw1[...])
  h = jax.nn.relu(jnp.dot(h, w2[...], preferred_element_type=jnp.float32) + bw2[...])
  y2_ref[...] = jnp.dot(h, wl2[...], preferred_element_type=jnp.float32)
  z2_ref[...] = jnp.dot(h, wr2[...], preferred_element_type=jnp.float32) + bl2[...]


def _tc2_body(sums_ref, cnts_ref, z2_ref, g2, beta2, w3, bw3, w4, bw4, out_ref):
  sum2 = sums_ref[0] + sums_ref[1]
  cnt = cnts_ref[0, :, 0:1] + cnts_ref[1, :, 0:1]
  mean2 = sum2 * (1.0 / jnp.maximum(cnt, 1.0))
  h = _ln_relu(mean2 + z2_ref[...], g2[...], beta2[...])
  h = jax.nn.relu(jnp.dot(h, w3[...], preferred_element_type=jnp.float32) + bw3[...])
  out_ref[...] = jnp.dot(h, w4[...], preferred_element_type=jnp.float32) + bw4[...]


def _row_spec(width):
  return pl.BlockSpec((ROW_BLK, width), lambda i: (i, 0))


def _part_spec(width):
  return pl.BlockSpec((NC, ROW_BLK, width), lambda i: (0, i, 0))


def _full_spec(shape):
  return pl.BlockSpec(shape, lambda i: tuple(0 for _ in shape))


def kernel(x, edge_index, Wl1, bl1, Wr1, g1, beta1, W1, bW1, W2, bW2,
           Wl2, bl2, Wr2, g2, beta2, W3, bW3, W4, bW4):
  f32 = jnp.float32
  x = x.astype(f32)
  src = edge_index[0].astype(jnp.int32)
  dst = edge_index[1].astype(jnp.int32)
  pad = EDGES_PAD - N_EDGES
  pad_idx = jnp.arange(pad, dtype=jnp.int32)
  src_pad = (pad_idx * 997) % N_NODES
  dst_pad = DUMMY + (pad_idx % (NPAD - N_NODES))
  src_r = jnp.concatenate([src, src_pad]).reshape(NS * K_TOTAL, CHUNK)
  dst_r = jnp.concatenate([dst, dst_pad]).reshape(NS * K_TOTAL, CHUNK)
  x_pad = jnp.zeros((NPAD, 128), f32).at[:N_NODES].set(x)
  zrow = jnp.zeros((CHUNK, 128), f32)
  ones = jnp.ones((CHUNK, 128), f32)

  cnts = _sc_cnt(dst_r, zrow, ones).reshape(NC, NPAD, 128)
  sums1 = _sc_agg(x_pad, src_r, dst_r, zrow).reshape(NC, NPAD, 128)

  grid = (GRID,)
  y2, z2 = pl.pallas_call(
      _tc1_body,
      grid=grid,
      in_specs=[
          _row_spec(128),
          _part_spec(128),
          _part_spec(128),
          _full_spec((128, 256)),
          _full_spec((1, 256)),
          _full_spec((128, 256)),
          _full_spec((1, 256)),
          _full_spec((1, 256)),
          _full_spec((256, 512)),
          _full_spec((1, 512)),
          _full_spec((512, 256)),
          _full_spec((1, 256)),
          _full_spec((256, 128)),
          _full_spec((256, 128)),
          _full_spec((1, 128)),
      ],
      out_specs=[_row_spec(128), _row_spec(128)],
      out_shape=[jax.ShapeDtypeStruct((NPAD, 128), f32),
                 jax.ShapeDtypeStruct((NPAD, 128), f32)],
  )(x_pad, sums1, cnts, Wl1.T, bl1[None, :], Wr1.T, g1[None, :],
    beta1[None, :], W1.T, bW1[None, :], W2.T, bW2[None, :], Wl2.T, Wr2.T,
    bl2[None, :])

  sums2 = _sc_agg(y2, src_r, dst_r, zrow).reshape(NC, NPAD, 128)

  (out,) = pl.pallas_call(
      _tc2_body,
      grid=grid,
      in_specs=[
          _part_spec(128),
          _part_spec(128),
          _row_spec(128),
          _full_spec((1, 128)),
          _full_spec((1, 128)),
          _full_spec((128, 256)),
          _full_spec((1, 256)),
          _full_spec((256, 128)),
          _full_spec((1, 128)),
      ],
      out_specs=[_row_spec(128)],
      out_shape=[jax.ShapeDtypeStruct((NPAD, 128), f32)],
  )(sums2, cnts, z2, g2[None, :], beta2[None, :], W3.T, bW3[None, :],
    W4.T, bW4[None, :])

  return out[:N_NODES]
